# baseline (speedup 1.0000x reference)
import functools

import jax
import jax.numpy as jnp
from jax import lax
from jax.experimental import pallas as pl
from jax.experimental.pallas import tpu as pltpu
from jax.experimental.pallas import tpu_sc as plsc

_P = [5024, 1257, 315, 80, 21]
_PP = [5120, 1280, 320, 128, 32]
_SP = 12
_B = 16
_FE = [3, 16, 16, 16, 32]
_FD = [32, 16, 16, 16, 3]
_LATENT = 128
_NW = 32


def _sc_gather(table, idx_pad):
    n_pad = idx_pad.shape[0]
    d = table.shape[1]
    n_per_w = n_pad // _NW
    c = n_per_w
    while c * d * 4 > 262144:
        c //= 2
    assert c % 8 == 0 and n_per_w % c == 0
    nchunk = n_per_w // c
    mesh = plsc.VectorSubcoreMesh(core_axis_name="c", subcore_axis_name="s")

    @functools.partial(
        pl.kernel,
        out_type=jax.ShapeDtypeStruct((n_pad, d), jnp.float32),
        mesh=mesh,
        scratch_types=[
            pltpu.VMEM((c,), jnp.int32),
            pltpu.VMEM((c, d), jnp.float32),
            pltpu.SemaphoreType.DMA,
        ],
    )
    def k(table_hbm, idx_hbm, out_hbm, idx_v, rows_v, sem):
        wid = lax.axis_index("s") * 2 + lax.axis_index("c")
        base = wid * n_per_w
        for g in range(nchunk):
            off = base + g * c
            pltpu.sync_copy(idx_hbm.at[pl.ds(off, c)], idx_v)
            pltpu.async_copy(table_hbm.at[idx_v], rows_v, sem).wait()
            pltpu.sync_copy(rows_v, out_hbm.at[pl.ds(off, c)])

    return k(table, idx_pad)


def _conv(g2, pp, wblk, bias_row, act, final_mask_limit=None):
    bf = g2.shape[1]
    bfo = wblk.shape[2]
    r = 512 if pp % 512 == 0 else (256 if pp % 256 == 0 else
                                   (pp if pp <= 256 else 160))
    gi = pp // r

    def body(g_ref, w_ref, b_ref, o_ref):
        j = pl.program_id(1)
        i = pl.program_id(0)
        contrib = jnp.dot(g_ref[...], w_ref[0],
                          preferred_element_type=jnp.float32)

        @pl.when(j == 0)
        def _():
            o_ref[...] = contrib

        @pl.when(j > 0)
        def _():
            o_ref[...] += contrib

        @pl.when(j == _SP - 1)
        def _():
            acc = o_ref[...] + b_ref[...]
            if act:
                acc = jnp.where(acc > 0, acc,
                                jnp.exp(jnp.minimum(acc, 0.0)) - 1.0)
            if final_mask_limit is not None:
                rowid = (lax.broadcasted_iota(jnp.int32, (r, 1), 0)
                         + i * r)
                acc = jnp.where(rowid < final_mask_limit, acc, 0.0)
            o_ref[...] = acc

    return pl.pallas_call(
        body,
        grid=(gi, _SP),
        in_specs=[
            pl.BlockSpec((r, bf), lambda i, j: (j * gi + i, 0)),
            pl.BlockSpec((1, bf, bfo), lambda i, j: (j, 0, 0)),
            pl.BlockSpec((1, bfo), lambda i, j: (0, 0)),
        ],
        out_specs=pl.BlockSpec((r, bfo), lambda i, j: (i, 0)),
        out_shape=jax.ShapeDtypeStruct((pp, bfo), jnp.float32),
    )(g2, wblk, bias_row)


def _pool(a, t, mask_limit):
    m, kk = a.shape
    kp, n = t.shape
    mb = min(256, m)
    kb = min(512, -(-kk // 128) * 128)
    gm = -(-m // mb)
    gk = -(-kk // kb)

    def body(a_ref, t_ref, o_ref):
        k = pl.program_id(1)
        ablk = a_ref[...]
        colid = lax.broadcasted_iota(jnp.int32, (mb, kb), 1) + k * kb
        ablk = jnp.where(colid < mask_limit, ablk, 0.0)
        tblk = t_ref[...]
        rowid = lax.broadcasted_iota(jnp.int32, (kb, n), 0) + k * kb
        tblk = jnp.where(rowid < mask_limit, tblk, 0.0)

        @pl.when(k == 0)
        def _():
            o_ref[...] = jnp.zeros_like(o_ref)

        o_ref[...] += jnp.dot(ablk, tblk, preferred_element_type=jnp.float32)

    return pl.pallas_call(
        body,
        grid=(gm, gk),
        in_specs=[
            pl.BlockSpec((mb, kb), lambda i, k: (i, k)),
            pl.BlockSpec((kb, n), lambda i, k: (k, 0)),
        ],
        out_specs=pl.BlockSpec((mb, n), lambda i, k: (i, 0)),
        out_shape=jax.ShapeDtypeStruct((m, n), jnp.float32),
    )(a, t)


def _fc(h4std, wfe, bfe_row, wfd, bfd_row):
    bsz, fin = h4std.shape
    fout = wfd.shape[1]

    def body(h_ref, a_ref, ab_ref, c_ref, cb_ref, o_ref):
        z = jnp.dot(h_ref[...], a_ref[...], preferred_element_type=jnp.float32)
        z = z + ab_ref[...]
        y = jnp.dot(z, c_ref[...], preferred_element_type=jnp.float32)
        o_ref[...] = y + cb_ref[...]

    return pl.pallas_call(
        body,
        out_shape=jax.ShapeDtypeStruct((bsz, fout), jnp.float32),
    )(h4std, wfe, bfe_row, wfd, bfd_row)


def _blockdiag(w, f_in, f_out):
    w3 = w.reshape(_SP, f_in, f_out)
    eye = jnp.eye(_B, dtype=jnp.float32)
    out = jnp.einsum('bB,jcd->jbcBd', eye, w3)
    return out.reshape(_SP, _B * f_in, _B * f_out)


def _packed_bias(b):
    return jnp.tile(b, _B)[None, :]


def _pad_idx(s, lvl):
    sp = jnp.zeros((_PP[lvl], _SP), jnp.int32).at[:_P[lvl]].set(s)
    return sp.T.reshape(-1)


def kernel(x, s0, s1, s2, s3, D0, D1, D2, D3, U0, U1, U2, U3,
           We0, be0, We1, be1, We2, be2, We3, be3,
           Wfe, bfe, Wfd, bfd,
           Wd0, bd0, Wd1, bd1, Wd2, bd2, Wd3, bd3):
    S = [s0, s1, s2, s3]
    D = [D0, D1, D2, D3]
    U = [U0, U1, U2, U3]
    We = [We0, We1, We2, We3]
    be = [be0, be1, be2, be3]
    Wd = [Wd0, Wd1, Wd2, Wd3]
    bd = [bd0, bd1, bd2, bd3]

    idx = [_pad_idx(S[i], i) for i in range(4)]

    h = x.transpose(1, 0, 2).reshape(_P[0], _B * _FE[0])
    h = jnp.pad(h, ((0, 0), (0, 128 - _B * _FE[0])))
    for i in range(4):
        g = _sc_gather(h, idx[i])
        wblk = _blockdiag(We[i], _FE[i], _FE[i + 1])
        if i == 0:
            wblk = jnp.pad(wblk, ((0, 0), (0, 128 - _B * _FE[0]), (0, 0)))
        t = _conv(g, _PP[i], wblk, _packed_bias(be[i]), act=True)
        h = _pool(D[i], t, _P[i] - 1)

    h4 = h.reshape(_P[4], _B, _FE[4]).transpose(1, 0, 2).reshape(_B, _P[4] * _FE[4])
    y5 = _fc(h4, Wfe, bfe[None, :], Wfd, bfd[None, :])
    y = y5.reshape(_B, _P[4], _FD[0]).transpose(1, 0, 2).reshape(_P[4], _B * _FD[0])

    for i in range(4):
        lvl = 3 - i
        limit = _P[lvl + 1] if i == 0 else _P[lvl + 1] - 1
        y = _pool(U[lvl], y, limit)
        g = _sc_gather(y, idx[lvl])
        wblk = _blockdiag(Wd[i], _FD[i], _FD[i + 1])
        final = i == 3
        y = _conv(g, _PP[lvl], wblk, _packed_bias(bd[i]), act=not final,
                  final_mask_limit=_P[0] - 1 if final else None)

    out = y[:_P[0]].reshape(_P[0], _B, _FD[4]).transpose(1, 0, 2)
    return out


# speedup vs baseline: 1.0916x; 1.0916x over previous
import functools

import jax
import jax.numpy as jnp
from jax import lax
from jax.experimental import pallas as pl
from jax.experimental.pallas import tpu as pltpu
from jax.experimental.pallas import tpu_sc as plsc

_P = [5024, 1257, 315, 80, 21]
_PP = [5120, 1280, 320, 128, 32]
_SP = 12
_B = 16
_FE = [3, 16, 16, 16, 32]
_FD = [32, 16, 16, 16, 3]
_LATENT = 128
_NW = 32


def _sc_gather(table, idx_pad):
    n_pad = idx_pad.shape[0]
    d = table.shape[1]
    n_per_w = n_pad // _NW
    c = n_per_w
    while c * d * 4 > 245760:
        c //= 2
    assert c % 8 == 0 and n_per_w % c == 0
    nchunk = n_per_w // c
    nbuf = 2 if nchunk > 1 else 1
    mesh = plsc.VectorSubcoreMesh(core_axis_name="c", subcore_axis_name="s")

    scratch = [pltpu.VMEM((n_per_w,), jnp.int32)]
    scratch += [pltpu.VMEM((c, d), jnp.float32) for _ in range(nbuf)]
    scratch += [pltpu.SemaphoreType.DMA for _ in range(2 * nbuf)]

    @functools.partial(
        pl.kernel,
        out_type=jax.ShapeDtypeStruct((n_pad, d), jnp.float32),
        mesh=mesh,
        scratch_types=scratch,
    )
    def k(table_hbm, idx_hbm, out_hbm, idx_v, *bs):
        bufs = bs[:nbuf]
        gsem = bs[nbuf:2 * nbuf]
        osem = bs[2 * nbuf:]
        wid = lax.axis_index("s") * 2 + lax.axis_index("c")
        base = wid * n_per_w
        pltpu.sync_copy(idx_hbm.at[pl.ds(base, n_per_w)], idx_v)

        def start_gather(g):
            b = g % nbuf
            return pltpu.async_copy(
                table_hbm.at[idx_v.at[pl.ds(g * c, c)]], bufs[b], gsem[b])

        gh = [None] * nchunk
        oh = [None] * nchunk
        gh[0] = start_gather(0)
        if nchunk > 1:
            gh[1] = start_gather(1)
        for g in range(nchunk):
            b = g % nbuf
            gh[g].wait()
            oh[g] = pltpu.async_copy(
                bufs[b], out_hbm.at[pl.ds(base + g * c, c)], osem[b])
            if g + 2 < nchunk:
                oh[g].wait()
                gh[g + 2] = start_gather(g + 2)
        for g in range(max(0, nchunk - 2), nchunk):
            oh[g].wait()

    return k(table, idx_pad)


def _conv(g2, pp, wblk, bias_row, act, final_mask_limit=None):
    bf = g2.shape[1]
    bfo = wblk.shape[2]
    r = 512 if pp % 512 == 0 else (256 if pp % 256 == 0 else
                                   (pp if pp <= 256 else 160))
    gi = pp // r

    def body(g_ref, w_ref, b_ref, o_ref):
        j = pl.program_id(1)
        i = pl.program_id(0)
        contrib = jnp.dot(g_ref[...], w_ref[0],
                          preferred_element_type=jnp.float32)

        @pl.when(j == 0)
        def _():
            o_ref[...] = contrib

        @pl.when(j > 0)
        def _():
            o_ref[...] += contrib

        @pl.when(j == _SP - 1)
        def _():
            acc = o_ref[...] + b_ref[...]
            if act:
                acc = jnp.where(acc > 0, acc,
                                jnp.exp(jnp.minimum(acc, 0.0)) - 1.0)
            if final_mask_limit is not None:
                rowid = (lax.broadcasted_iota(jnp.int32, (r, 1), 0)
                         + i * r)
                acc = jnp.where(rowid < final_mask_limit, acc, 0.0)
            o_ref[...] = acc

    return pl.pallas_call(
        body,
        grid=(gi, _SP),
        in_specs=[
            pl.BlockSpec((r, bf), lambda i, j: (j * gi + i, 0)),
            pl.BlockSpec((1, bf, bfo), lambda i, j: (j, 0, 0)),
            pl.BlockSpec((1, bfo), lambda i, j: (0, 0)),
        ],
        out_specs=pl.BlockSpec((r, bfo), lambda i, j: (i, 0)),
        out_shape=jax.ShapeDtypeStruct((pp, bfo), jnp.float32),
    )(g2, wblk, bias_row)


def _pool(a, t, mask_limit):
    m, kk = a.shape
    kp, n = t.shape
    mb = min(256, m)
    kb = min(512, -(-kk // 128) * 128)
    gm = -(-m // mb)
    gk = -(-kk // kb)

    def body(a_ref, t_ref, o_ref):
        k = pl.program_id(1)
        ablk = a_ref[...]
        colid = lax.broadcasted_iota(jnp.int32, (mb, kb), 1) + k * kb
        ablk = jnp.where(colid < mask_limit, ablk, 0.0)
        tblk = t_ref[...]
        rowid = lax.broadcasted_iota(jnp.int32, (kb, n), 0) + k * kb
        tblk = jnp.where(rowid < mask_limit, tblk, 0.0)

        @pl.when(k == 0)
        def _():
            o_ref[...] = jnp.zeros_like(o_ref)

        o_ref[...] += jnp.dot(ablk, tblk, preferred_element_type=jnp.float32)

    return pl.pallas_call(
        body,
        grid=(gm, gk),
        in_specs=[
            pl.BlockSpec((mb, kb), lambda i, k: (i, k)),
            pl.BlockSpec((kb, n), lambda i, k: (k, 0)),
        ],
        out_specs=pl.BlockSpec((mb, n), lambda i, k: (i, 0)),
        out_shape=jax.ShapeDtypeStruct((m, n), jnp.float32),
    )(a, t)


def _fc(h4std, wfe, bfe_row, wfd, bfd_row):
    bsz, fin = h4std.shape
    fout = wfd.shape[1]

    def body(h_ref, a_ref, ab_ref, c_ref, cb_ref, o_ref):
        z = jnp.dot(h_ref[...], a_ref[...], preferred_element_type=jnp.float32)
        z = z + ab_ref[...]
        y = jnp.dot(z, c_ref[...], preferred_element_type=jnp.float32)
        o_ref[...] = y + cb_ref[...]

    return pl.pallas_call(
        body,
        out_shape=jax.ShapeDtypeStruct((bsz, fout), jnp.float32),
    )(h4std, wfe, bfe_row, wfd, bfd_row)


def _blockdiag(w, f_in, f_out, pad_to=None):
    bf = pad_to or _B * f_in
    bfo = _B * f_out
    w3 = w.reshape(_SP, f_in, f_out)

    def body(w_ref, o_ref):
        wj = w_ref[0]
        rows = jnp.concatenate([wj] * _B, axis=0)
        tile = jnp.concatenate([rows] * _B, axis=1)
        rid = lax.broadcasted_iota(jnp.int32, (_B * f_in, bfo), 0)
        cid = lax.broadcasted_iota(jnp.int32, (_B * f_in, bfo), 1)
        blk = jnp.where(rid // f_in == cid // f_out, tile, 0.0)
        if bf > _B * f_in:
            blk = jnp.concatenate(
                [blk, jnp.zeros((bf - _B * f_in, bfo), jnp.float32)], axis=0)
        o_ref[0] = blk

    return pl.pallas_call(
        body,
        grid=(_SP,),
        in_specs=[pl.BlockSpec((1, f_in, f_out), lambda j: (j, 0, 0))],
        out_specs=pl.BlockSpec((1, bf, bfo), lambda j: (j, 0, 0)),
        out_shape=jax.ShapeDtypeStruct((_SP, bf, bfo), jnp.float32),
    )(w3)


def _pack_x(x):
    p0 = x.shape[1]
    r = 512
    gi = -(-p0 // r)

    def body(x_ref, o_ref):
        parts = [x_ref[b] for b in range(_B)]
        parts.append(jnp.zeros((r, 128 - _B * _FE[0]), jnp.float32))
        o_ref[...] = jnp.concatenate(parts, axis=1)

    return pl.pallas_call(
        body,
        grid=(gi,),
        in_specs=[pl.BlockSpec((_B, r, _FE[0]), lambda i: (0, i, 0))],
        out_specs=pl.BlockSpec((r, 128), lambda i: (i, 0)),
        out_shape=jax.ShapeDtypeStruct((p0, 128), jnp.float32),
    )(x)


def _packed_bias(b):
    return jnp.tile(b, _B)[None, :]


def _pad_idx(s, lvl):
    sp = jnp.zeros((_PP[lvl], _SP), jnp.int32).at[:_P[lvl]].set(s)
    return sp.T.reshape(-1)


def kernel(x, s0, s1, s2, s3, D0, D1, D2, D3, U0, U1, U2, U3,
           We0, be0, We1, be1, We2, be2, We3, be3,
           Wfe, bfe, Wfd, bfd,
           Wd0, bd0, Wd1, bd1, Wd2, bd2, Wd3, bd3):
    S = [s0, s1, s2, s3]
    D = [D0, D1, D2, D3]
    U = [U0, U1, U2, U3]
    We = [We0, We1, We2, We3]
    be = [be0, be1, be2, be3]
    Wd = [Wd0, Wd1, Wd2, Wd3]
    bd = [bd0, bd1, bd2, bd3]

    idx = [_pad_idx(S[i], i) for i in range(4)]

    h = _pack_x(x)
    for i in range(4):
        g = _sc_gather(h, idx[i])
        wblk = _blockdiag(We[i], _FE[i], _FE[i + 1],
                          pad_to=128 if i == 0 else None)
        t = _conv(g, _PP[i], wblk, _packed_bias(be[i]), act=True)
        h = _pool(D[i], t, _P[i] - 1)

    h4 = h.reshape(_P[4], _B, _FE[4]).transpose(1, 0, 2).reshape(_B, _P[4] * _FE[4])
    y5 = _fc(h4, Wfe, bfe[None, :], Wfd, bfd[None, :])
    y = y5.reshape(_B, _P[4], _FD[0]).transpose(1, 0, 2).reshape(_P[4], _B * _FD[0])

    for i in range(4):
        lvl = 3 - i
        limit = _P[lvl + 1] if i == 0 else _P[lvl + 1] - 1
        y = _pool(U[lvl], y, limit)
        g = _sc_gather(y, idx[lvl])
        wblk = _blockdiag(Wd[i], _FD[i], _FD[i + 1])
        final = i == 3
        y = _conv(g, _PP[lvl], wblk, _packed_bias(bd[i]), act=not final,
                  final_mask_limit=_P[0] - 1 if final else None)

    out = y[:_P[0]].reshape(_P[0], _B, _FD[4]).transpose(1, 0, 2)
    return out


# speedup vs baseline: 1.1737x; 1.0752x over previous
import functools

import jax
import jax.numpy as jnp
from jax import lax
from jax.experimental import pallas as pl
from jax.experimental.pallas import tpu as pltpu
from jax.experimental.pallas import tpu_sc as plsc

_P = [5024, 1257, 315, 80, 21]
_PP = [5120, 1280, 320, 128, 32]
_SP = 12
_B = 16
_FE = [3, 16, 16, 16, 32]
_FD = [32, 16, 16, 16, 3]
_LATENT = 128
_NW = 32


def _sc_gather(table, idx_pad):
    n_pad = idx_pad.shape[0]
    d = table.shape[1]
    dt = table.dtype
    n_per_w = n_pad // _NW
    c = n_per_w
    while c * d * dt.itemsize > 245760:
        c //= 2
    assert c % 8 == 0 and n_per_w % c == 0
    nchunk = n_per_w // c
    nbuf = 2 if nchunk > 1 else 1
    mesh = plsc.VectorSubcoreMesh(core_axis_name="c", subcore_axis_name="s")

    scratch = [pltpu.VMEM((n_per_w,), jnp.int32)]
    scratch += [pltpu.VMEM((c, d), dt) for _ in range(nbuf)]
    scratch += [pltpu.SemaphoreType.DMA for _ in range(2 * nbuf)]

    @functools.partial(
        pl.kernel,
        out_type=jax.ShapeDtypeStruct((n_pad, d), dt),
        mesh=mesh,
        scratch_types=scratch,
    )
    def k(table_hbm, idx_hbm, out_hbm, idx_v, *bs):
        bufs = bs[:nbuf]
        gsem = bs[nbuf:2 * nbuf]
        osem = bs[2 * nbuf:]
        wid = lax.axis_index("s") * 2 + lax.axis_index("c")
        base = wid * n_per_w
        pltpu.sync_copy(idx_hbm.at[pl.ds(base, n_per_w)], idx_v)

        def start_gather(g):
            b = g % nbuf
            return pltpu.async_copy(
                table_hbm.at[idx_v.at[pl.ds(g * c, c)]], bufs[b], gsem[b])

        gh = [None] * nchunk
        oh = [None] * nchunk
        gh[0] = start_gather(0)
        if nchunk > 1:
            gh[1] = start_gather(1)
        for g in range(nchunk):
            b = g % nbuf
            gh[g].wait()
            oh[g] = pltpu.async_copy(
                bufs[b], out_hbm.at[pl.ds(base + g * c, c)], osem[b])
            if g + 2 < nchunk:
                oh[g].wait()
                gh[g + 2] = start_gather(g + 2)
        for g in range(max(0, nchunk - 2), nchunk):
            oh[g].wait()

    return k(table, idx_pad)


def _pack_pair(x):
    h = x.shape[1] // 2
    ua = jax.lax.bitcast_convert_type(x[:, :h], jnp.uint32)
    ub = jax.lax.bitcast_convert_type(x[:, h:], jnp.uint32)
    ra = (ua + 0x7FFF + ((ua >> 16) & 1)) & jnp.uint32(0xFFFF0000)
    rb = (ub + 0x7FFF + ((ub >> 16) & 1)) & jnp.uint32(0xFFFF0000)
    return jax.lax.bitcast_convert_type(ra | (rb >> 16), jnp.float32)


def _unpack_pair(p):
    u = jax.lax.bitcast_convert_type(p, jnp.uint32)
    va = jax.lax.bitcast_convert_type(u & jnp.uint32(0xFFFF0000), jnp.float32)
    vb = jax.lax.bitcast_convert_type(u << 16, jnp.float32)
    return jnp.concatenate([va, vb], axis=1)


def _conv(g2, pp, wblk, bias_row, act, final_mask_limit=None, packed_in=False):
    bf = g2.shape[1]
    wrows = wblk.shape[1]
    bfo = wblk.shape[2]
    r = 512 if pp % 512 == 0 else (256 if pp % 256 == 0 else
                                   (pp if pp <= 256 else 160))
    gi = pp // r

    def body(g_ref, w_ref, b_ref, o_ref):
        j = pl.program_id(1)
        i = pl.program_id(0)
        gblk = g_ref[...]
        if packed_in:
            gblk = _unpack_pair(gblk)
        contrib = jnp.dot(gblk, w_ref[0],
                          preferred_element_type=jnp.float32)

        @pl.when(j == 0)
        def _():
            o_ref[...] = contrib

        @pl.when(j > 0)
        def _():
            o_ref[...] += contrib

        @pl.when(j == _SP - 1)
        def _():
            acc = o_ref[...] + b_ref[...]
            if act:
                acc = jnp.where(acc > 0, acc,
                                jnp.exp(jnp.minimum(acc, 0.0)) - 1.0)
            if final_mask_limit is not None:
                rowid = (lax.broadcasted_iota(jnp.int32, (r, 1), 0)
                         + i * r)
                acc = jnp.where(rowid < final_mask_limit, acc, 0.0)
            o_ref[...] = acc

    return pl.pallas_call(
        body,
        grid=(gi, _SP),
        in_specs=[
            pl.BlockSpec((r, bf), lambda i, j: (j * gi + i, 0)),
            pl.BlockSpec((1, wrows, bfo), lambda i, j: (j, 0, 0)),
            pl.BlockSpec((1, bfo), lambda i, j: (0, 0)),
        ],
        out_specs=pl.BlockSpec((r, bfo), lambda i, j: (i, 0)),
        out_shape=jax.ShapeDtypeStruct((pp, bfo), jnp.float32),
    )(g2, wblk, bias_row)


def _pool(a, t, mask_limit, pack=False):
    m, kk = a.shape
    kp, n = t.shape
    no = n // 2 if pack else n
    mb = min(256, m)
    kb = min(512, -(-kk // 128) * 128)
    gm = -(-m // mb)
    gk = -(-kk // kb)

    def body(a_ref, t_ref, o_ref, acc_ref):
        k = pl.program_id(1)
        ablk = a_ref[...]
        colid = lax.broadcasted_iota(jnp.int32, (mb, kb), 1) + k * kb
        ablk = jnp.where(colid < mask_limit, ablk, 0.0)
        tblk = t_ref[...]
        rowid = lax.broadcasted_iota(jnp.int32, (kb, n), 0) + k * kb
        tblk = jnp.where(rowid < mask_limit, tblk, 0.0)

        @pl.when(k == 0)
        def _():
            acc_ref[...] = jnp.zeros_like(acc_ref)

        acc_ref[...] += jnp.dot(ablk, tblk, preferred_element_type=jnp.float32)

        @pl.when(k == gk - 1)
        def _():
            acc = acc_ref[...]
            o_ref[...] = _pack_pair(acc) if pack else acc

    return pl.pallas_call(
        body,
        grid=(gm, gk),
        in_specs=[
            pl.BlockSpec((mb, kb), lambda i, k: (i, k)),
            pl.BlockSpec((kb, n), lambda i, k: (k, 0)),
        ],
        out_specs=pl.BlockSpec((mb, no), lambda i, k: (i, 0)),
        out_shape=jax.ShapeDtypeStruct((m, no), jnp.float32),
        scratch_shapes=[pltpu.VMEM((mb, n), jnp.float32)],
    )(a, t)


def _fc(h4std, wfe, bfe_row, wfd, bfd_row):
    bsz, fin = h4std.shape
    fout = wfd.shape[1]

    def body(h_ref, a_ref, ab_ref, c_ref, cb_ref, o_ref):
        z = jnp.dot(h_ref[...], a_ref[...], preferred_element_type=jnp.float32)
        z = z + ab_ref[...]
        y = jnp.dot(z, c_ref[...], preferred_element_type=jnp.float32)
        o_ref[...] = y + cb_ref[...]

    return pl.pallas_call(
        body,
        out_shape=jax.ShapeDtypeStruct((bsz, fout), jnp.float32),
    )(h4std, wfe, bfe_row, wfd, bfd_row)


def _blockdiag(w, f_in, f_out, pad_to=None, dtype=jnp.float32):
    bf = pad_to or _B * f_in
    bfo = _B * f_out
    w3 = w.reshape(_SP, f_in, f_out)

    def body(w_ref, o_ref):
        wj = w_ref[0]
        rows = jnp.concatenate([wj] * _B, axis=0)
        tile = jnp.concatenate([rows] * _B, axis=1)
        rid = lax.broadcasted_iota(jnp.int32, (_B * f_in, bfo), 0)
        cid = lax.broadcasted_iota(jnp.int32, (_B * f_in, bfo), 1)
        blk = jnp.where(rid // f_in == cid // f_out, tile, 0.0)
        if bf > _B * f_in:
            blk = jnp.concatenate(
                [blk, jnp.zeros((bf - _B * f_in, bfo), jnp.float32)], axis=0)
        o_ref[0] = blk.astype(dtype)

    return pl.pallas_call(
        body,
        grid=(_SP,),
        in_specs=[pl.BlockSpec((1, f_in, f_out), lambda j: (j, 0, 0))],
        out_specs=pl.BlockSpec((1, bf, bfo), lambda j: (j, 0, 0)),
        out_shape=jax.ShapeDtypeStruct((_SP, bf, bfo), dtype),
    )(w3)


def _pack_x(x):
    p0 = x.shape[1]
    r = 512
    gi = -(-p0 // r)

    def body(x_ref, o_ref):
        parts = [x_ref[b] for b in range(_B)]
        parts.append(jnp.zeros((r, 128 - _B * _FE[0]), jnp.float32))
        o_ref[...] = jnp.concatenate(parts, axis=1)

    return pl.pallas_call(
        body,
        grid=(gi,),
        in_specs=[pl.BlockSpec((_B, r, _FE[0]), lambda i: (0, i, 0))],
        out_specs=pl.BlockSpec((r, 128), lambda i: (i, 0)),
        out_shape=jax.ShapeDtypeStruct((p0, 128), jnp.float32),
    )(x)


def _packed_bias(b):
    return jnp.tile(b, _B)[None, :]


def _pad_idx(s, lvl):
    sp = jnp.zeros((_PP[lvl], _SP), jnp.int32).at[:_P[lvl]].set(s)
    return sp.T.reshape(-1)


def kernel(x, s0, s1, s2, s3, D0, D1, D2, D3, U0, U1, U2, U3,
           We0, be0, We1, be1, We2, be2, We3, be3,
           Wfe, bfe, Wfd, bfd,
           Wd0, bd0, Wd1, bd1, Wd2, bd2, Wd3, bd3):
    S = [s0, s1, s2, s3]
    D = [D0, D1, D2, D3]
    U = [U0, U1, U2, U3]
    We = [We0, We1, We2, We3]
    be = [be0, be1, be2, be3]
    Wd = [Wd0, Wd1, Wd2, Wd3]
    bd = [bd0, bd1, bd2, bd3]

    idx = [_pad_idx(S[i], i) for i in range(4)]

    h = _pack_x(x)
    for i in range(4):
        g = _sc_gather(h, idx[i])
        wblk = _blockdiag(We[i], _FE[i], _FE[i + 1],
                          pad_to=128 if i == 0 else None)
        t = _conv(g, _PP[i], wblk, _packed_bias(be[i]), act=True,
                  packed_in=(i == 1))
        h = _pool(D[i], t, _P[i] - 1, pack=(i == 0))

    h4 = h.reshape(_P[4], _B, _FE[4]).transpose(1, 0, 2).reshape(_B, _P[4] * _FE[4])
    y5 = _fc(h4, Wfe, bfe[None, :], Wfd, bfd[None, :])
    y = y5.reshape(_B, _P[4], _FD[0]).transpose(1, 0, 2).reshape(_P[4], _B * _FD[0])

    for i in range(4):
        lvl = 3 - i
        limit = _P[lvl + 1] if i == 0 else _P[lvl + 1] - 1
        pk = lvl <= 1
        y = _pool(U[lvl], y, limit, pack=pk)
        g = _sc_gather(y, idx[lvl])
        wblk = _blockdiag(Wd[i], _FD[i], _FD[i + 1])
        final = i == 3
        y = _conv(g, _PP[lvl], wblk, _packed_bias(bd[i]), act=not final,
                  final_mask_limit=_P[0] - 1 if final else None, packed_in=pk)

    out = y[:_P[0]].reshape(_P[0], _B, _FD[4]).transpose(1, 0, 2)
    return out


# speedup vs baseline: 1.3128x; 1.1185x over previous
import functools

import jax
import jax.numpy as jnp
from jax import lax
from jax.experimental import pallas as pl
from jax.experimental.pallas import tpu as pltpu
from jax.experimental.pallas import tpu_sc as plsc

_P = [5024, 1257, 315, 80, 21]
_PP = [5120, 1280, 320, 128, 32]
_SP = 12
_B = 16
_FE = [3, 16, 16, 16, 32]
_FD = [32, 16, 16, 16, 3]
_LATENT = 128
_NW = 32


def _sc_gather(table, idx_pad):
    n_pad = idx_pad.shape[0]
    d = table.shape[1]
    dt = table.dtype
    n_per_w = n_pad // _NW
    c = n_per_w
    while c * d * dt.itemsize > 245760:
        c //= 2
    assert c % 8 == 0 and n_per_w % c == 0
    nchunk = n_per_w // c
    nbuf = 2 if nchunk > 1 else 1
    mesh = plsc.VectorSubcoreMesh(core_axis_name="c", subcore_axis_name="s")

    scratch = [pltpu.VMEM((n_per_w,), jnp.int32)]
    scratch += [pltpu.VMEM((c, d), dt) for _ in range(nbuf)]
    scratch += [pltpu.SemaphoreType.DMA for _ in range(2 * nbuf)]

    @functools.partial(
        pl.kernel,
        out_type=jax.ShapeDtypeStruct((n_pad, d), dt),
        mesh=mesh,
        scratch_types=scratch,
    )
    def k(table_hbm, idx_hbm, out_hbm, idx_v, *bs):
        bufs = bs[:nbuf]
        gsem = bs[nbuf:2 * nbuf]
        osem = bs[2 * nbuf:]
        wid = lax.axis_index("s") * 2 + lax.axis_index("c")
        base = wid * n_per_w
        pltpu.sync_copy(idx_hbm.at[pl.ds(base, n_per_w)], idx_v)

        def start_gather(g):
            b = g % nbuf
            return pltpu.async_copy(
                table_hbm.at[idx_v.at[pl.ds(g * c, c)]], bufs[b], gsem[b])

        gh = [None] * nchunk
        oh = [None] * nchunk
        gh[0] = start_gather(0)
        if nchunk > 1:
            gh[1] = start_gather(1)
        for g in range(nchunk):
            b = g % nbuf
            gh[g].wait()
            oh[g] = pltpu.async_copy(
                bufs[b], out_hbm.at[pl.ds(base + g * c, c)], osem[b])
            if g + 2 < nchunk:
                oh[g].wait()
                gh[g + 2] = start_gather(g + 2)
        for g in range(max(0, nchunk - 2), nchunk):
            oh[g].wait()

    return k(table, idx_pad)


def _pack_pair(x):
    h = x.shape[1] // 2
    ua = jax.lax.bitcast_convert_type(x[:, :h], jnp.uint32)
    ub = jax.lax.bitcast_convert_type(x[:, h:], jnp.uint32)
    ra = (ua + 0x7FFF + ((ua >> 16) & 1)) & jnp.uint32(0xFFFF0000)
    rb = (ub + 0x7FFF + ((ub >> 16) & 1)) & jnp.uint32(0xFFFF0000)
    return jax.lax.bitcast_convert_type(ra | (rb >> 16), jnp.float32)


def _unpack_pair(p):
    u = jax.lax.bitcast_convert_type(p, jnp.uint32)
    va = jax.lax.bitcast_convert_type(u & jnp.uint32(0xFFFF0000), jnp.float32)
    vb = jax.lax.bitcast_convert_type(u << 16, jnp.float32)
    return jnp.concatenate([va, vb], axis=1)


def _conv(g2, pp, wblk, bias_row, act, final_mask_limit=None, packed_in=False):
    bf = g2.shape[1]
    wrows = wblk.shape[1]
    bfo = wblk.shape[2]
    r = 512 if pp % 512 == 0 else (256 if pp % 256 == 0 else
                                   (pp if pp <= 256 else 160))
    gi = pp // r

    def body(g_ref, w_ref, b_ref, o_ref):
        j = pl.program_id(0)
        i = pl.program_id(1)
        gblk = g_ref[...]
        if packed_in:
            gblk = _unpack_pair(gblk)
        contrib = jnp.dot(gblk, w_ref[0],
                          preferred_element_type=jnp.float32)
        sl = pl.ds(i * r, r)

        @pl.when(j == 0)
        def _():
            o_ref[sl, :] = contrib

        @pl.when(j > 0)
        def _():
            o_ref[sl, :] += contrib

        @pl.when(j == _SP - 1)
        def _():
            acc = o_ref[sl, :] + b_ref[...]
            if act:
                acc = jnp.where(acc > 0, acc,
                                jnp.exp(jnp.minimum(acc, 0.0)) - 1.0)
            if final_mask_limit is not None:
                rowid = (lax.broadcasted_iota(jnp.int32, (r, 1), 0)
                         + i * r)
                acc = jnp.where(rowid < final_mask_limit, acc, 0.0)
            o_ref[sl, :] = acc

    return pl.pallas_call(
        body,
        grid=(_SP, gi),
        in_specs=[
            pl.BlockSpec((r, bf), lambda j, i: (j * gi + i, 0)),
            pl.BlockSpec((1, wrows, bfo), lambda j, i: (j, 0, 0)),
            pl.BlockSpec((1, bfo), lambda j, i: (0, 0)),
        ],
        out_specs=pl.BlockSpec((pp, bfo), lambda j, i: (0, 0)),
        out_shape=jax.ShapeDtypeStruct((pp, bfo), jnp.float32),
    )(g2, wblk, bias_row)


def _pool(a, t, mask_limit, pack=False, a_transposed=False):
    if a_transposed:
        kk, m = a.shape
    else:
        m, kk = a.shape
    kp, n = t.shape
    no = n // 2 if pack else n
    mb = min(256, m)
    kb = min(512, -(-kk // 128) * 128)
    gm = -(-m // mb)
    nk = -(-kk // kb)
    kkp = nk * kb

    def body(a_ref, t_ref, o_ref):
        acc = jnp.zeros((mb, n), jnp.float32)
        for k in range(nk):
            sl = pl.ds(k * kb, kb)
            tblk = t_ref[sl, :]
            rowid = lax.broadcasted_iota(jnp.int32, (kb, n), 0) + k * kb
            tblk = jnp.where(rowid < mask_limit, tblk, 0.0)
            if a_transposed:
                ablk = a_ref[sl, :]
                kid = lax.broadcasted_iota(jnp.int32, (kb, mb), 0) + k * kb
                ablk = jnp.where(kid < mask_limit, ablk, 0.0)
                acc = acc + lax.dot_general(
                    ablk, tblk, (((0,), (0,)), ((), ())),
                    preferred_element_type=jnp.float32)
            else:
                ablk = a_ref[:, sl]
                kid = lax.broadcasted_iota(jnp.int32, (mb, kb), 1) + k * kb
                ablk = jnp.where(kid < mask_limit, ablk, 0.0)
                acc = acc + jnp.dot(ablk, tblk,
                                    preferred_element_type=jnp.float32)
        o_ref[...] = _pack_pair(acc) if pack else acc

    if a_transposed:
        a_spec = pl.BlockSpec((kkp, mb), lambda i: (0, i))
    else:
        a_spec = pl.BlockSpec((mb, kkp), lambda i: (i, 0))
    return pl.pallas_call(
        body,
        grid=(gm,),
        in_specs=[
            a_spec,
            pl.BlockSpec((kkp, n), lambda i: (0, 0)),
        ],
        out_specs=pl.BlockSpec((mb, no), lambda i: (i, 0)),
        out_shape=jax.ShapeDtypeStruct((m, no), jnp.float32),
    )(a, t)


def _fc(h4std, wfe, bfe_row, wfd, bfd_row):
    bsz, fin = h4std.shape
    fout = wfd.shape[1]

    def body(h_ref, a_ref, ab_ref, c_ref, cb_ref, o_ref):
        z = jnp.dot(h_ref[...], a_ref[...], preferred_element_type=jnp.float32)
        z = z + ab_ref[...]
        y = jnp.dot(z, c_ref[...], preferred_element_type=jnp.float32)
        o_ref[...] = y + cb_ref[...]

    return pl.pallas_call(
        body,
        out_shape=jax.ShapeDtypeStruct((bsz, fout), jnp.float32),
    )(h4std, wfe, bfe_row, wfd, bfd_row)


def _blockdiag(w, f_in, f_out, pad_to=None, dtype=jnp.float32):
    bf = pad_to or _B * f_in
    bfo = _B * f_out
    w3 = w.reshape(_SP, f_in, f_out)

    def body(w_ref, o_ref):
        wj = w_ref[0]
        rows = jnp.concatenate([wj] * _B, axis=0)
        tile = jnp.concatenate([rows] * _B, axis=1)
        rid = lax.broadcasted_iota(jnp.int32, (_B * f_in, bfo), 0)
        cid = lax.broadcasted_iota(jnp.int32, (_B * f_in, bfo), 1)
        blk = jnp.where(rid // f_in == cid // f_out, tile, 0.0)
        if bf > _B * f_in:
            blk = jnp.concatenate(
                [blk, jnp.zeros((bf - _B * f_in, bfo), jnp.float32)], axis=0)
        o_ref[0] = blk.astype(dtype)

    return pl.pallas_call(
        body,
        grid=(_SP,),
        in_specs=[pl.BlockSpec((1, f_in, f_out), lambda j: (j, 0, 0))],
        out_specs=pl.BlockSpec((1, bf, bfo), lambda j: (j, 0, 0)),
        out_shape=jax.ShapeDtypeStruct((_SP, bf, bfo), dtype),
    )(w3)


def _pack_x(x):
    p0 = x.shape[1]
    r = 512
    gi = -(-p0 // r)

    def body(x_ref, o_ref):
        parts = [x_ref[b] for b in range(_B)]
        parts.append(jnp.zeros((r, 128 - _B * _FE[0]), jnp.float32))
        o_ref[...] = jnp.concatenate(parts, axis=1)

    return pl.pallas_call(
        body,
        grid=(gi,),
        in_specs=[pl.BlockSpec((_B, r, _FE[0]), lambda i: (0, i, 0))],
        out_specs=pl.BlockSpec((r, 128), lambda i: (i, 0)),
        out_shape=jax.ShapeDtypeStruct((p0, 128), jnp.float32),
    )(x)


def _packed_bias(b):
    return jnp.tile(b, _B)[None, :]


def _pad_idx(s, lvl):
    sp = jnp.zeros((_PP[lvl], _SP), jnp.int32).at[:_P[lvl]].set(s)
    return sp.T.reshape(-1)


def kernel(x, s0, s1, s2, s3, D0, D1, D2, D3, U0, U1, U2, U3,
           We0, be0, We1, be1, We2, be2, We3, be3,
           Wfe, bfe, Wfd, bfd,
           Wd0, bd0, Wd1, bd1, Wd2, bd2, Wd3, bd3):
    S = [s0, s1, s2, s3]
    D = [D0, D1, D2, D3]
    U = [U0, U1, U2, U3]
    We = [We0, We1, We2, We3]
    be = [be0, be1, be2, be3]
    Wd = [Wd0, Wd1, Wd2, Wd3]
    bd = [bd0, bd1, bd2, bd3]

    idx = [_pad_idx(S[i], i) for i in range(4)]

    h = _pack_x(x)
    for i in range(4):
        g = _sc_gather(h, idx[i])
        wblk = _blockdiag(We[i], _FE[i], _FE[i + 1],
                          pad_to=128 if i == 0 else None)
        t = _conv(g, _PP[i], wblk, _packed_bias(be[i]), act=True,
                  packed_in=(i == 1))
        if i == 0:
            h = _pool(D[0].T, t, _P[0] - 1, pack=True, a_transposed=True)
        else:
            h = _pool(D[i], t, _P[i] - 1)

    h4 = h.reshape(_P[4], _B, _FE[4]).transpose(1, 0, 2).reshape(_B, _P[4] * _FE[4])
    y5 = _fc(h4, Wfe, bfe[None, :], Wfd, bfd[None, :])
    y = y5.reshape(_B, _P[4], _FD[0]).transpose(1, 0, 2).reshape(_P[4], _B * _FD[0])

    for i in range(4):
        lvl = 3 - i
        limit = _P[lvl + 1] if i == 0 else _P[lvl + 1] - 1
        pk = lvl <= 1
        y = _pool(U[lvl], y, limit, pack=pk)
        g = _sc_gather(y, idx[lvl])
        wblk = _blockdiag(Wd[i], _FD[i], _FD[i + 1])
        final = i == 3
        y = _conv(g, _PP[lvl], wblk, _packed_bias(bd[i]), act=not final,
                  final_mask_limit=_P[0] - 1 if final else None, packed_in=pk)

    out = y[:_P[0]].reshape(_P[0], _B, _FD[4]).transpose(1, 0, 2)
    return out


# speedup vs baseline: 1.3211x; 1.0063x over previous
import functools

import jax
import jax.numpy as jnp
from jax import lax
from jax.experimental import pallas as pl
from jax.experimental.pallas import tpu as pltpu
from jax.experimental.pallas import tpu_sc as plsc

_P = [5024, 1257, 315, 80, 21]
_PP = [5120, 1280, 320, 128, 32]
_SP = 12
_B = 16
_FE = [3, 16, 16, 16, 32]
_FD = [32, 16, 16, 16, 3]
_LATENT = 128
_NW = 32


def _sc_gather(table, idx_pad):
    n_pad = idx_pad.shape[0]
    d = table.shape[1]
    dt = table.dtype
    n_per_w = n_pad // _NW
    c = n_per_w
    while c * d * dt.itemsize > 245760:
        c //= 2
    assert c % 8 == 0 and n_per_w % c == 0
    nchunk = n_per_w // c
    nbuf = 2 if nchunk > 1 else 1
    mesh = plsc.VectorSubcoreMesh(core_axis_name="c", subcore_axis_name="s")

    scratch = [pltpu.VMEM((n_per_w,), jnp.int32)]
    scratch += [pltpu.VMEM((c, d), dt) for _ in range(nbuf)]
    scratch += [pltpu.SemaphoreType.DMA for _ in range(2 * nbuf)]

    @functools.partial(
        pl.kernel,
        out_type=jax.ShapeDtypeStruct((n_pad, d), dt),
        mesh=mesh,
        scratch_types=scratch,
    )
    def k(table_hbm, idx_hbm, out_hbm, idx_v, *bs):
        bufs = bs[:nbuf]
        gsem = bs[nbuf:2 * nbuf]
        osem = bs[2 * nbuf:]
        wid = lax.axis_index("s") * 2 + lax.axis_index("c")
        base = wid * n_per_w
        pltpu.sync_copy(idx_hbm.at[pl.ds(base, n_per_w)], idx_v)

        def start_gather(g):
            b = g % nbuf
            return pltpu.async_copy(
                table_hbm.at[idx_v.at[pl.ds(g * c, c)]], bufs[b], gsem[b])

        gh = [None] * nchunk
        oh = [None] * nchunk
        gh[0] = start_gather(0)
        if nchunk > 1:
            gh[1] = start_gather(1)
        for g in range(nchunk):
            b = g % nbuf
            gh[g].wait()
            oh[g] = pltpu.async_copy(
                bufs[b], out_hbm.at[pl.ds(base + g * c, c)], osem[b])
            if g + 2 < nchunk:
                oh[g].wait()
                gh[g + 2] = start_gather(g + 2)
        for g in range(max(0, nchunk - 2), nchunk):
            oh[g].wait()

    return k(table, idx_pad)


def _pack_pair(x):
    h = x.shape[1] // 2
    ua = jax.lax.bitcast_convert_type(x[:, :h], jnp.uint32)
    ub = jax.lax.bitcast_convert_type(x[:, h:], jnp.uint32)
    ra = (ua + 0x7FFF + ((ua >> 16) & 1)) & jnp.uint32(0xFFFF0000)
    rb = (ub + 0x7FFF + ((ub >> 16) & 1)) & jnp.uint32(0xFFFF0000)
    return jax.lax.bitcast_convert_type(ra | (rb >> 16), jnp.float32)


def _unpack_pair(p):
    u = jax.lax.bitcast_convert_type(p, jnp.uint32)
    va = jax.lax.bitcast_convert_type(u & jnp.uint32(0xFFFF0000), jnp.float32)
    vb = jax.lax.bitcast_convert_type(u << 16, jnp.float32)
    return jnp.concatenate([va, vb], axis=1)


def _conv(g2, pp, wblk, bias_row, act, final_mask_limit=None, packed_in=False,
          mxu_bf16=False):
    bf = g2.shape[1]
    wrows = wblk.shape[1]
    bfo = wblk.shape[2]
    r = 512 if pp % 512 == 0 else (256 if pp % 256 == 0 else
                                   (pp if pp <= 256 else 160))
    gi = pp // r

    def body(g_ref, w_ref, b_ref, o_ref):
        j = pl.program_id(0)
        i = pl.program_id(1)
        gblk = g_ref[...]
        if packed_in:
            gblk = _unpack_pair(gblk)
        if mxu_bf16:
            gblk = gblk.astype(jnp.bfloat16)
        contrib = jnp.dot(gblk, w_ref[0],
                          preferred_element_type=jnp.float32)
        sl = pl.ds(i * r, r)

        @pl.when(j == 0)
        def _():
            o_ref[sl, :] = contrib

        @pl.when(j > 0)
        def _():
            o_ref[sl, :] += contrib

        @pl.when(j == _SP - 1)
        def _():
            acc = o_ref[sl, :] + b_ref[...]
            if act:
                acc = jnp.where(acc > 0, acc,
                                jnp.exp(jnp.minimum(acc, 0.0)) - 1.0)
            if final_mask_limit is not None:
                rowid = (lax.broadcasted_iota(jnp.int32, (r, 1), 0)
                         + i * r)
                acc = jnp.where(rowid < final_mask_limit, acc, 0.0)
            o_ref[sl, :] = acc

    return pl.pallas_call(
        body,
        grid=(_SP, gi),
        in_specs=[
            pl.BlockSpec((r, bf), lambda j, i: (j * gi + i, 0)),
            pl.BlockSpec((1, wrows, bfo), lambda j, i: (j, 0, 0)),
            pl.BlockSpec((1, bfo), lambda j, i: (0, 0)),
        ],
        out_specs=pl.BlockSpec((pp, bfo), lambda j, i: (0, 0)),
        out_shape=jax.ShapeDtypeStruct((pp, bfo), jnp.float32),
    )(g2, wblk, bias_row)


def _pool(a, t, mask_limit, pack=False, a_transposed=False):
    if a_transposed:
        kk, m = a.shape
    else:
        m, kk = a.shape
    kp, n = t.shape
    no = n // 2 if pack else n
    mb = min(256, m)
    kb = min(512, -(-kk // 128) * 128)
    gm = -(-m // mb)
    nk = -(-kk // kb)
    kkp = nk * kb

    def body(a_ref, t_ref, o_ref):
        acc = jnp.zeros((mb, n), jnp.float32)
        for k in range(nk):
            sl = pl.ds(k * kb, kb)
            tblk = t_ref[sl, :]
            rowid = lax.broadcasted_iota(jnp.int32, (kb, n), 0) + k * kb
            tblk = jnp.where(rowid < mask_limit, tblk, 0.0)
            if a_transposed:
                ablk = a_ref[sl, :]
                kid = lax.broadcasted_iota(jnp.int32, (kb, mb), 0) + k * kb
                ablk = jnp.where(kid < mask_limit, ablk, 0.0)
                acc = acc + lax.dot_general(
                    ablk, tblk, (((0,), (0,)), ((), ())),
                    preferred_element_type=jnp.float32)
            else:
                ablk = a_ref[:, sl]
                kid = lax.broadcasted_iota(jnp.int32, (mb, kb), 1) + k * kb
                ablk = jnp.where(kid < mask_limit, ablk, 0.0)
                acc = acc + jnp.dot(ablk, tblk,
                                    preferred_element_type=jnp.float32)
        o_ref[...] = _pack_pair(acc) if pack else acc

    if a_transposed:
        a_spec = pl.BlockSpec((kkp, mb), lambda i: (0, i))
    else:
        a_spec = pl.BlockSpec((mb, kkp), lambda i: (i, 0))
    return pl.pallas_call(
        body,
        grid=(gm,),
        in_specs=[
            a_spec,
            pl.BlockSpec((kkp, n), lambda i: (0, 0)),
        ],
        out_specs=pl.BlockSpec((mb, no), lambda i: (i, 0)),
        out_shape=jax.ShapeDtypeStruct((m, no), jnp.float32),
    )(a, t)


def _fc(h4std, wfe, bfe_row, wfd, bfd_row):
    bsz, fin = h4std.shape
    fout = wfd.shape[1]

    def body(h_ref, a_ref, ab_ref, c_ref, cb_ref, o_ref):
        z = jnp.dot(h_ref[...], a_ref[...], preferred_element_type=jnp.float32)
        z = z + ab_ref[...]
        y = jnp.dot(z, c_ref[...], preferred_element_type=jnp.float32)
        o_ref[...] = y + cb_ref[...]

    return pl.pallas_call(
        body,
        out_shape=jax.ShapeDtypeStruct((bsz, fout), jnp.float32),
    )(h4std, wfe, bfe_row, wfd, bfd_row)


def _blockdiag(w, f_in, f_out, pad_to=None, dtype=jnp.float32):
    bf = pad_to or _B * f_in
    bfo = _B * f_out
    w3 = w.reshape(_SP, f_in, f_out)

    def body(w_ref, o_ref):
        wj = w_ref[0]
        rows = jnp.concatenate([wj] * _B, axis=0)
        tile = jnp.concatenate([rows] * _B, axis=1)
        rid = lax.broadcasted_iota(jnp.int32, (_B * f_in, bfo), 0)
        cid = lax.broadcasted_iota(jnp.int32, (_B * f_in, bfo), 1)
        blk = jnp.where(rid // f_in == cid // f_out, tile, 0.0)
        if bf > _B * f_in:
            blk = jnp.concatenate(
                [blk, jnp.zeros((bf - _B * f_in, bfo), jnp.float32)], axis=0)
        o_ref[0] = blk.astype(dtype)

    return pl.pallas_call(
        body,
        grid=(_SP,),
        in_specs=[pl.BlockSpec((1, f_in, f_out), lambda j: (j, 0, 0))],
        out_specs=pl.BlockSpec((1, bf, bfo), lambda j: (j, 0, 0)),
        out_shape=jax.ShapeDtypeStruct((_SP, bf, bfo), dtype),
    )(w3)


def _pack_x(x):
    p0 = x.shape[1]
    r = 512
    gi = -(-p0 // r)

    def body(x_ref, o_ref):
        parts = [x_ref[b] for b in range(_B)]
        parts.append(jnp.zeros((r, 128 - _B * _FE[0]), jnp.float32))
        o_ref[...] = jnp.concatenate(parts, axis=1)

    return pl.pallas_call(
        body,
        grid=(gi,),
        in_specs=[pl.BlockSpec((_B, r, _FE[0]), lambda i: (0, i, 0))],
        out_specs=pl.BlockSpec((r, 128), lambda i: (i, 0)),
        out_shape=jax.ShapeDtypeStruct((p0, 128), jnp.float32),
    )(x)


def _packed_bias(b):
    return jnp.tile(b, _B)[None, :]


def _pad_idx(s, lvl):
    sp = jnp.zeros((_PP[lvl], _SP), jnp.int32).at[:_P[lvl]].set(s)
    return sp.T.reshape(-1)


def kernel(x, s0, s1, s2, s3, D0, D1, D2, D3, U0, U1, U2, U3,
           We0, be0, We1, be1, We2, be2, We3, be3,
           Wfe, bfe, Wfd, bfd,
           Wd0, bd0, Wd1, bd1, Wd2, bd2, Wd3, bd3):
    S = [s0, s1, s2, s3]
    D = [D0, D1, D2, D3]
    U = [U0, U1, U2, U3]
    We = [We0, We1, We2, We3]
    be = [be0, be1, be2, be3]
    Wd = [Wd0, Wd1, Wd2, Wd3]
    bd = [bd0, bd1, bd2, bd3]

    idx = [_pad_idx(S[i], i) for i in range(4)]

    h = _pack_x(x)
    for i in range(4):
        g = _sc_gather(h, idx[i])
        bfconv = i <= 1
        wblk = _blockdiag(We[i], _FE[i], _FE[i + 1],
                          pad_to=128 if i == 0 else None,
                          dtype=jnp.bfloat16 if bfconv else jnp.float32)
        t = _conv(g, _PP[i], wblk, _packed_bias(be[i]), act=True,
                  packed_in=(i == 1), mxu_bf16=bfconv)
        if i == 0:
            h = _pool(D[0].T, t, _P[0] - 1, pack=True, a_transposed=True)
        else:
            h = _pool(D[i], t, _P[i] - 1)

    h4 = h.reshape(_P[4], _B, _FE[4]).transpose(1, 0, 2).reshape(_B, _P[4] * _FE[4])
    y5 = _fc(h4, Wfe, bfe[None, :], Wfd, bfd[None, :])
    y = y5.reshape(_B, _P[4], _FD[0]).transpose(1, 0, 2).reshape(_P[4], _B * _FD[0])

    for i in range(4):
        lvl = 3 - i
        limit = _P[lvl + 1] if i == 0 else _P[lvl + 1] - 1
        pk = lvl <= 1
        y = _pool(U[lvl], y, limit, pack=pk)
        g = _sc_gather(y, idx[lvl])
        wblk = _blockdiag(Wd[i], _FD[i], _FD[i + 1],
                          dtype=jnp.bfloat16 if pk else jnp.float32)
        final = i == 3
        y = _conv(g, _PP[lvl], wblk, _packed_bias(bd[i]), act=not final,
                  final_mask_limit=_P[0] - 1 if final else None, packed_in=pk,
                  mxu_bf16=pk)

    out = y[:_P[0]].reshape(_P[0], _B, _FD[4]).transpose(1, 0, 2)
    return out


# speedup vs baseline: 1.3807x; 1.0452x over previous
import functools

import jax
import jax.numpy as jnp
from jax import lax
from jax.experimental import pallas as pl
from jax.experimental.pallas import tpu as pltpu
from jax.experimental.pallas import tpu_sc as plsc

_P = [5024, 1257, 315, 80, 21]
_PP = [5120, 1280, 320, 128, 32]
_SP = 12
_B = 16
_FE = [3, 16, 16, 16, 32]
_FD = [32, 16, 16, 16, 3]
_LATENT = 128
_NW = 32


def _sc_gather(table, idx_pad):
    n_pad = idx_pad.shape[0]
    d = table.shape[1]
    dt = table.dtype
    n_per_w = n_pad // _NW
    c = n_per_w
    while c * d * dt.itemsize > 245760:
        c //= 2
    assert c % 8 == 0 and n_per_w % c == 0
    nchunk = n_per_w // c
    nbuf = 2 if nchunk > 1 else 1
    mesh = plsc.VectorSubcoreMesh(core_axis_name="c", subcore_axis_name="s")

    scratch = [pltpu.VMEM((n_per_w,), jnp.int32)]
    scratch += [pltpu.VMEM((c, d), dt) for _ in range(nbuf)]
    scratch += [pltpu.SemaphoreType.DMA for _ in range(2 * nbuf)]

    @functools.partial(
        pl.kernel,
        out_type=jax.ShapeDtypeStruct((n_pad, d), dt),
        mesh=mesh,
        scratch_types=scratch,
    )
    def k(table_hbm, idx_hbm, out_hbm, idx_v, *bs):
        bufs = bs[:nbuf]
        gsem = bs[nbuf:2 * nbuf]
        osem = bs[2 * nbuf:]
        wid = lax.axis_index("s") * 2 + lax.axis_index("c")
        base = wid * n_per_w
        pltpu.sync_copy(idx_hbm.at[pl.ds(base, n_per_w)], idx_v)

        def start_gather(g):
            b = g % nbuf
            return pltpu.async_copy(
                table_hbm.at[idx_v.at[pl.ds(g * c, c)]], bufs[b], gsem[b])

        gh = [None] * nchunk
        oh = [None] * nchunk
        gh[0] = start_gather(0)
        if nchunk > 1:
            gh[1] = start_gather(1)
        for g in range(nchunk):
            b = g % nbuf
            gh[g].wait()
            oh[g] = pltpu.async_copy(
                bufs[b], out_hbm.at[pl.ds(base + g * c, c)], osem[b])
            if g + 2 < nchunk:
                oh[g].wait()
                gh[g + 2] = start_gather(g + 2)
        for g in range(max(0, nchunk - 2), nchunk):
            oh[g].wait()

    return k(table, idx_pad)


def _pack_pair(x):
    h = x.shape[1] // 2
    ua = jax.lax.bitcast_convert_type(x[:, :h], jnp.uint32)
    ub = jax.lax.bitcast_convert_type(x[:, h:], jnp.uint32)
    ra = (ua + 0x7FFF + ((ua >> 16) & 1)) & jnp.uint32(0xFFFF0000)
    rb = (ub + 0x7FFF + ((ub >> 16) & 1)) & jnp.uint32(0xFFFF0000)
    return jax.lax.bitcast_convert_type(ra | (rb >> 16), jnp.float32)


def _unpack_pair(p):
    u = jax.lax.bitcast_convert_type(p, jnp.uint32)
    va = jax.lax.bitcast_convert_type(u & jnp.uint32(0xFFFF0000), jnp.float32)
    vb = jax.lax.bitcast_convert_type(u << 16, jnp.float32)
    return jnp.concatenate([va, vb], axis=1)


def _conv(g2, pp, wblk, bias_row, act, final_mask_limit=None, packed_in=False,
          mxu_bf16=False):
    bf = g2.shape[1]
    wrows = wblk.shape[1]
    bfo = wblk.shape[2]
    r = 512 if pp % 512 == 0 else (256 if pp % 256 == 0 else
                                   (pp if pp <= 256 else 160))
    gi = pp // r

    def body(g_ref, w_ref, b_ref, o_ref):
        j = pl.program_id(0)
        i = pl.program_id(1)
        gblk = g_ref[...]
        if packed_in:
            gblk = _unpack_pair(gblk)
        if mxu_bf16:
            gblk = gblk.astype(jnp.bfloat16)
        contrib = jnp.dot(gblk, w_ref[0],
                          preferred_element_type=jnp.float32)
        sl = pl.ds(i * r, r)

        @pl.when(j == 0)
        def _():
            o_ref[sl, :] = contrib

        @pl.when(j > 0)
        def _():
            o_ref[sl, :] += contrib

        @pl.when(j == _SP - 1)
        def _():
            acc = o_ref[sl, :] + b_ref[...]
            if act:
                acc = jnp.where(acc > 0, acc,
                                jnp.exp(jnp.minimum(acc, 0.0)) - 1.0)
            if final_mask_limit is not None:
                rowid = (lax.broadcasted_iota(jnp.int32, (r, 1), 0)
                         + i * r)
                acc = jnp.where(rowid < final_mask_limit, acc, 0.0)
            o_ref[sl, :] = acc

    return pl.pallas_call(
        body,
        grid=(_SP, gi),
        in_specs=[
            pl.BlockSpec((r, bf), lambda j, i: (j * gi + i, 0)),
            pl.BlockSpec((1, wrows, bfo), lambda j, i: (j, 0, 0)),
            pl.BlockSpec((1, bfo), lambda j, i: (0, 0)),
        ],
        out_specs=pl.BlockSpec((pp, bfo), lambda j, i: (0, 0)),
        out_shape=jax.ShapeDtypeStruct((pp, bfo), jnp.float32),
    )(g2, wblk, bias_row)


def _pool(a, t, mask_limit, pack=False, a_transposed=False):
    if a_transposed:
        kk, m = a.shape
    else:
        m, kk = a.shape
    kp, n = t.shape
    no = n // 2 if pack else n
    mb = min(256, m)
    kb = min(512, -(-kk // 128) * 128)
    gm = -(-m // mb)
    nk = -(-kk // kb)
    kkp = nk * kb

    def body(a_ref, t_ref, o_ref):
        acc = jnp.zeros((mb, n), jnp.float32)
        for k in range(nk):
            sl = pl.ds(k * kb, kb)
            tblk = t_ref[sl, :]
            rowid = lax.broadcasted_iota(jnp.int32, (kb, n), 0) + k * kb
            tblk = jnp.where(rowid < mask_limit, tblk, 0.0)
            if a_transposed:
                ablk = a_ref[sl, :]
                kid = lax.broadcasted_iota(jnp.int32, (kb, mb), 0) + k * kb
                ablk = jnp.where(kid < mask_limit, ablk, 0.0)
                acc = acc + lax.dot_general(
                    ablk, tblk, (((0,), (0,)), ((), ())),
                    preferred_element_type=jnp.float32)
            else:
                ablk = a_ref[:, sl]
                kid = lax.broadcasted_iota(jnp.int32, (mb, kb), 1) + k * kb
                ablk = jnp.where(kid < mask_limit, ablk, 0.0)
                acc = acc + jnp.dot(ablk, tblk,
                                    preferred_element_type=jnp.float32)
        o_ref[...] = _pack_pair(acc) if pack else acc

    if a_transposed:
        a_spec = pl.BlockSpec((kkp, mb), lambda i: (0, i))
    else:
        a_spec = pl.BlockSpec((mb, kkp), lambda i: (i, 0))
    return pl.pallas_call(
        body,
        grid=(gm,),
        in_specs=[
            a_spec,
            pl.BlockSpec((kkp, n), lambda i: (0, 0)),
        ],
        out_specs=pl.BlockSpec((mb, no), lambda i: (i, 0)),
        out_shape=jax.ShapeDtypeStruct((m, no), jnp.float32),
    )(a, t)


def _fc(h4std, wfe, bfe_row, wfd, bfd_row):
    bsz, fin = h4std.shape
    fout = wfd.shape[1]

    def body(h_ref, a_ref, ab_ref, c_ref, cb_ref, o_ref):
        z = jnp.dot(h_ref[...], a_ref[...], preferred_element_type=jnp.float32)
        z = z + ab_ref[...]
        y = jnp.dot(z, c_ref[...], preferred_element_type=jnp.float32)
        o_ref[...] = y + cb_ref[...]

    return pl.pallas_call(
        body,
        out_shape=jax.ShapeDtypeStruct((bsz, fout), jnp.float32),
    )(h4std, wfe, bfe_row, wfd, bfd_row)


def _blockdiag(w, f_in, f_out, pad_to=None, dtype=jnp.float32):
    bf = pad_to or _B * f_in
    bfo = _B * f_out
    w3 = w.reshape(_SP, f_in, f_out)

    def body(w_ref, o_ref):
        wj = w_ref[0]
        rows = jnp.concatenate([wj] * _B, axis=0)
        tile = jnp.concatenate([rows] * _B, axis=1)
        rid = lax.broadcasted_iota(jnp.int32, (_B * f_in, bfo), 0)
        cid = lax.broadcasted_iota(jnp.int32, (_B * f_in, bfo), 1)
        blk = jnp.where(rid // f_in == cid // f_out, tile, 0.0)
        if bf > _B * f_in:
            blk = jnp.concatenate(
                [blk, jnp.zeros((bf - _B * f_in, bfo), jnp.float32)], axis=0)
        o_ref[0] = blk.astype(dtype)

    return pl.pallas_call(
        body,
        grid=(_SP,),
        in_specs=[pl.BlockSpec((1, f_in, f_out), lambda j: (j, 0, 0))],
        out_specs=pl.BlockSpec((1, bf, bfo), lambda j: (j, 0, 0)),
        out_shape=jax.ShapeDtypeStruct((_SP, bf, bfo), dtype),
    )(w3)


def _pack_x(x):
    p0 = x.shape[1]
    x3 = x.transpose(2, 0, 1)
    r = 512
    gi = -(-p0 // r)

    def body(x_ref, o_ref):
        xb = x_ref[...].transpose(1, 0, 2).reshape(_B * _FE[0], r)
        flat = xb.transpose(1, 0)
        pad = jnp.zeros((r, 128 - _B * _FE[0]), jnp.float32)
        o_ref[...] = jnp.concatenate([flat, pad], axis=1)

    return pl.pallas_call(
        body,
        grid=(gi,),
        in_specs=[pl.BlockSpec((_FE[0], _B, r), lambda i: (0, 0, i))],
        out_specs=pl.BlockSpec((r, 128), lambda i: (i, 0)),
        out_shape=jax.ShapeDtypeStruct((p0, 128), jnp.float32),
    )(x3)


def _packed_bias(b):
    return jnp.tile(b, _B)[None, :]


def _pad_idx(s, lvl):
    sp = jnp.zeros((_PP[lvl], _SP), jnp.int32).at[:_P[lvl]].set(s)
    return sp.T.reshape(-1)


def kernel(x, s0, s1, s2, s3, D0, D1, D2, D3, U0, U1, U2, U3,
           We0, be0, We1, be1, We2, be2, We3, be3,
           Wfe, bfe, Wfd, bfd,
           Wd0, bd0, Wd1, bd1, Wd2, bd2, Wd3, bd3):
    S = [s0, s1, s2, s3]
    D = [D0, D1, D2, D3]
    U = [U0, U1, U2, U3]
    We = [We0, We1, We2, We3]
    be = [be0, be1, be2, be3]
    Wd = [Wd0, Wd1, Wd2, Wd3]
    bd = [bd0, bd1, bd2, bd3]

    idx = [_pad_idx(S[i], i) for i in range(4)]

    h = _pack_x(x)
    for i in range(4):
        g = _sc_gather(h, idx[i])
        bfconv = i <= 1
        wblk = _blockdiag(We[i], _FE[i], _FE[i + 1],
                          pad_to=128 if i == 0 else None,
                          dtype=jnp.bfloat16 if bfconv else jnp.float32)
        t = _conv(g, _PP[i], wblk, _packed_bias(be[i]), act=True,
                  packed_in=(i == 1), mxu_bf16=bfconv)
        if i == 0:
            h = _pool(D[0].T, t, _P[0] - 1, pack=True, a_transposed=True)
        else:
            h = _pool(D[i], t, _P[i] - 1)

    h4 = h.reshape(_P[4], _B, _FE[4]).transpose(1, 0, 2).reshape(_B, _P[4] * _FE[4])
    y5 = _fc(h4, Wfe, bfe[None, :], Wfd, bfd[None, :])
    y = y5.reshape(_B, _P[4], _FD[0]).transpose(1, 0, 2).reshape(_P[4], _B * _FD[0])

    for i in range(4):
        lvl = 3 - i
        limit = _P[lvl + 1] if i == 0 else _P[lvl + 1] - 1
        pk = lvl <= 1
        y = _pool(U[lvl], y, limit, pack=pk)
        g = _sc_gather(y, idx[lvl])
        wblk = _blockdiag(Wd[i], _FD[i], _FD[i + 1],
                          dtype=jnp.bfloat16 if pk else jnp.float32)
        final = i == 3
        y = _conv(g, _PP[lvl], wblk, _packed_bias(bd[i]), act=not final,
                  final_mask_limit=_P[0] - 1 if final else None, packed_in=pk,
                  mxu_bf16=pk)

    out = y[:_P[0]].reshape(_P[0], _B, _FD[4]).transpose(1, 0, 2)
    return out
